# Initial kernel scaffold; baseline (speedup 1.0000x reference)
import jax
import jax.numpy as jnp
from jax.experimental import pallas as pl


def kernel(X, adj, W_w, W_b, S):
    raise NotImplementedError("write your pallas kernel here")


# speedup vs baseline: 3.5250x; 3.5250x over previous
import functools

import jax
import jax.numpy as jnp
from jax import lax
from jax.experimental import pallas as pl
from jax.experimental.pallas import tpu as pltpu

N = 4096
F = 256
BM = 512
BN = 512
NI = N // BM
NJ = N // BN
NEG = -1e30


def _body(x_ref, adj_ref, w_ref, b_ref, s_ref, out_ref,
          xp_ref, as_ref, ad_ref, m_ref, l_ref, acc_ref):
    i = pl.program_id(0)
    j = pl.program_id(1)

    @pl.when((i == 0) & (j == 0))
    def _init_xp():
        xp = lax.dot_general(x_ref[...], w_ref[...],
                             (((1,), (1,)), ((), ())),
                             preferred_element_type=jnp.float32)
        xp = xp + b_ref[0:1, :]
        xp_ref[...] = xp
        s_src = s_ref[0:1, 0:F]
        s_dst = s_ref[0:1, F:2 * F]
        as_ref[...] = lax.dot_general(xp, s_src, (((1,), (1,)), ((), ())),
                                      preferred_element_type=jnp.float32)
        ad_ref[...] = lax.dot_general(s_dst, xp, (((1,), (1,)), ((), ())),
                                      preferred_element_type=jnp.float32)

    @pl.when(j == 0)
    def _reset():
        m_ref[...] = jnp.full((BM, 1), NEG, dtype=jnp.float32)
        l_ref[...] = jnp.zeros((BM, 1), dtype=jnp.float32)
        acc_ref[...] = jnp.zeros((BM, F), dtype=jnp.float32)

    a_i = as_ref[pl.ds(i * BM, BM), :]
    a_j = ad_ref[:, pl.ds(j * BN, BN)]
    e = a_i + a_j
    e = jnp.where(e >= 0, e, 0.01 * e)

    cnt = (adj_ref[...] != 0).astype(jnp.float32)
    r = lax.broadcasted_iota(jnp.int32, (BM, BN), 0) + i * BM
    c = lax.broadcasted_iota(jnp.int32, (BM, BN), 1) + j * BN
    cnt = cnt + (r == c).astype(jnp.float32)
    valid = cnt > 0.0

    m_cur = jnp.max(jnp.where(valid, e, NEG), axis=1, keepdims=True)
    m_prev = m_ref[...]
    m_new = jnp.maximum(m_prev, m_cur)
    alpha = jnp.exp(m_prev - m_new)
    p = cnt * jnp.exp(jnp.where(valid, e - m_new, NEG))
    l_ref[...] = alpha * l_ref[...] + jnp.sum(p, axis=1, keepdims=True)
    m_ref[...] = m_new
    acc_ref[...] = alpha * acc_ref[...] + lax.dot_general(
        p, xp_ref[pl.ds(j * BN, BN), :], (((1,), (0,)), ((), ())),
        preferred_element_type=jnp.float32)

    @pl.when(j == NJ - 1)
    def _finalize():
        out_ref[...] = jax.nn.sigmoid(acc_ref[...] / l_ref[...])


@functools.partial(jax.jit, static_argnames=("interpret",))
def kernel(X, adj, W_w, W_b, S, interpret=False):
    b2 = W_b.reshape(1, F)
    s2 = S.reshape(1, 2 * F)
    return pl.pallas_call(
        _body,
        grid=(NI, NJ),
        in_specs=[
            pl.BlockSpec((N, F), lambda i, j: (0, 0)),
            pl.BlockSpec((BM, BN), lambda i, j: (i, j)),
            pl.BlockSpec((F, F), lambda i, j: (0, 0)),
            pl.BlockSpec((1, F), lambda i, j: (0, 0)),
            pl.BlockSpec((1, 2 * F), lambda i, j: (0, 0)),
        ],
        out_specs=pl.BlockSpec((BM, F), lambda i, j: (i, 0)),
        out_shape=jax.ShapeDtypeStruct((N, F), jnp.float32),
        scratch_shapes=[
            pltpu.VMEM((N, F), jnp.float32),
            pltpu.VMEM((N, 1), jnp.float32),
            pltpu.VMEM((1, N), jnp.float32),
            pltpu.VMEM((BM, 1), jnp.float32),
            pltpu.VMEM((BM, 1), jnp.float32),
            pltpu.VMEM((BM, F), jnp.float32),
        ],
        compiler_params=pltpu.CompilerParams(
            dimension_semantics=("arbitrary", "arbitrary"),
        ),
        interpret=interpret,
    )(X, adj, W_w, b2, s2)


# speedup vs baseline: 4.2010x; 1.1918x over previous
import functools

import jax
import jax.numpy as jnp
from jax import lax
from jax.experimental import pallas as pl
from jax.experimental.pallas import tpu as pltpu

N = 4096
F = 256
BM = 512
BN = 512
NI = N // BM
NJ = N // BN


def _body(x_ref, adj_ref, w_ref, b_ref, s_ref, out_ref,
          xp_ref, a_ref, c_ref, br_ref, dr_ref, bc_ref, dc_ref,
          l_ref, acc_ref):
    i = pl.program_id(0)
    j = pl.program_id(1)

    @pl.when((i == 0) & (j == 0))
    def _init():
        xp = lax.dot_general(x_ref[...], w_ref[...],
                             (((1,), (1,)), ((), ())),
                             preferred_element_type=jnp.float32)
        xp = xp + b_ref[0:1, :]
        xp_ref[...] = xp
        s_src = s_ref[0:1, 0:F]
        s_dst = s_ref[0:1, F:2 * F]
        s_col = lax.dot_general(xp, s_src, (((1,), (1,)), ((), ())),
                                preferred_element_type=jnp.float32)
        d_col = lax.dot_general(xp, s_dst, (((1,), (1,)), ((), ())),
                                preferred_element_type=jnp.float32)
        d_row = lax.dot_general(s_dst, xp, (((1,), (1,)), ((), ())),
                                preferred_element_type=jnp.float32)
        dmax = jnp.max(d_row)
        x_sm = s_col + dmax
        c_i = jnp.maximum(x_sm, 0.01 * x_sm)
        a_ref[...] = jnp.exp(x_sm - c_i)
        c_ref[...] = jnp.exp(0.01 * x_sm - c_i)
        br_ref[...] = jnp.exp(d_row - dmax)
        dr_ref[...] = jnp.exp(0.01 * (d_row - dmax))
        bc_ref[...] = jnp.exp(d_col - dmax)
        dc_ref[...] = jnp.exp(0.01 * (d_col - dmax))

    @pl.when(j == 0)
    def _reset():
        l_ref[...] = jnp.zeros((BM, 1), dtype=jnp.float32)
        acc_ref[...] = jnp.zeros((BM, F), dtype=jnp.float32)

    a_i = a_ref[pl.ds(i * BM, BM), :]
    c_i = c_ref[pl.ds(i * BM, BM), :]
    b_j = br_ref[:, pl.ds(j * BN, BN)]
    d_j = dr_ref[:, pl.ds(j * BN, BN)]
    q = jnp.maximum(a_i * b_j, c_i * d_j)
    p = q * adj_ref[...].astype(jnp.float32)
    l_ref[...] += jnp.sum(p, axis=1, keepdims=True)
    acc_ref[...] += lax.dot_general(
        p, xp_ref[pl.ds(j * BN, BN), :], (((1,), (0,)), ((), ())),
        preferred_element_type=jnp.float32)

    @pl.when(i == j)
    def _diag():
        bd = bc_ref[pl.ds(i * BM, BM), :]
        dd = dc_ref[pl.ds(i * BM, BM), :]
        dv = jnp.maximum(a_i * bd, c_i * dd)
        l_ref[...] += dv
        acc_ref[...] += dv * xp_ref[pl.ds(i * BM, BM), :]

    @pl.when(j == NJ - 1)
    def _finalize():
        out_ref[...] = jax.nn.sigmoid(acc_ref[...] / l_ref[...])


@functools.partial(jax.jit, static_argnames=("interpret",))
def kernel(X, adj, W_w, W_b, S, interpret=False):
    b2 = W_b.reshape(1, F)
    s2 = S.reshape(1, 2 * F)
    return pl.pallas_call(
        _body,
        grid=(NI, NJ),
        in_specs=[
            pl.BlockSpec((N, F), lambda i, j: (0, 0)),
            pl.BlockSpec((BM, BN), lambda i, j: (i, j)),
            pl.BlockSpec((F, F), lambda i, j: (0, 0)),
            pl.BlockSpec((1, F), lambda i, j: (0, 0)),
            pl.BlockSpec((1, 2 * F), lambda i, j: (0, 0)),
        ],
        out_specs=pl.BlockSpec((BM, F), lambda i, j: (i, 0)),
        out_shape=jax.ShapeDtypeStruct((N, F), jnp.float32),
        scratch_shapes=[
            pltpu.VMEM((N, F), jnp.float32),
            pltpu.VMEM((N, 1), jnp.float32),
            pltpu.VMEM((N, 1), jnp.float32),
            pltpu.VMEM((1, N), jnp.float32),
            pltpu.VMEM((1, N), jnp.float32),
            pltpu.VMEM((N, 1), jnp.float32),
            pltpu.VMEM((N, 1), jnp.float32),
            pltpu.VMEM((BM, 1), jnp.float32),
            pltpu.VMEM((BM, F), jnp.float32),
        ],
        compiler_params=pltpu.CompilerParams(
            dimension_semantics=("arbitrary", "arbitrary"),
        ),
        interpret=interpret,
    )(X, adj, W_w, b2, s2)
